# initial kernel scaffold (unmeasured)
import jax
import jax.numpy as jnp
from jax import lax
from jax.experimental import pallas as pl
from jax.experimental.pallas import tpu as pltpu


def kernel(
    x,
):
    def body(*refs):
        pass

    out_shape = jax.ShapeDtypeStruct(..., jnp.float32)
    return pl.pallas_call(body, out_shape=out_shape)(...)


# baseline (device time: 105769 ns/iter reference)
import jax
import jax.numpy as jnp
from jax import lax
from jax.experimental import pallas as pl
from jax.experimental.pallas import tpu as pltpu

N_DEV = 8


def kernel(x):
    m_per, n = x.shape
    half = m_per // 2
    n_hops = N_DEV - 1

    def body(x_ref, out_ref, send_f, recv_f, send_b, recv_b):
        my = lax.axis_index("i")
        right = (my + 1) % N_DEV
        left = (my + N_DEV - 1) % N_DEV

        barrier_sem = pltpu.get_barrier_semaphore()
        for nbr in (left, right):
            pl.semaphore_signal(
                barrier_sem, inc=1,
                device_id=(nbr,), device_id_type=pl.DeviceIdType.MESH,
            )
        pl.semaphore_wait(barrier_sem, 2)

        out_ref[pl.ds(my * m_per, m_per), :] = x_ref[:, :].astype(out_ref.dtype)

        for h in range(n_hops):
            o_f = (my + N_DEV - h) % N_DEV
            fwd = pltpu.make_async_remote_copy(
                src_ref=out_ref.at[pl.ds(o_f * m_per, half), :],
                dst_ref=out_ref.at[pl.ds(o_f * m_per, half), :],
                send_sem=send_f.at[h],
                recv_sem=recv_f.at[h],
                device_id=(right,),
                device_id_type=pl.DeviceIdType.MESH,
            )
            o_b = (my + h) % N_DEV
            bwd = pltpu.make_async_remote_copy(
                src_ref=out_ref.at[pl.ds(o_b * m_per + half, half), :],
                dst_ref=out_ref.at[pl.ds(o_b * m_per + half, half), :],
                send_sem=send_b.at[h],
                recv_sem=recv_b.at[h],
                device_id=(left,),
                device_id_type=pl.DeviceIdType.MESH,
            )
            fwd.start()
            bwd.start()
            fwd.wait()
            bwd.wait()

    out_shape = jax.ShapeDtypeStruct((N_DEV * m_per, n), jnp.bfloat16)
    return pl.pallas_call(
        body,
        out_shape=out_shape,
        in_specs=[pl.BlockSpec(memory_space=pltpu.VMEM)],
        out_specs=pl.BlockSpec(memory_space=pltpu.VMEM),
        scratch_shapes=[
            pltpu.SemaphoreType.DMA((n_hops,)),
            pltpu.SemaphoreType.DMA((n_hops,)),
            pltpu.SemaphoreType.DMA((n_hops,)),
            pltpu.SemaphoreType.DMA((n_hops,)),
        ],
        compiler_params=pltpu.CompilerParams(collective_id=0),
    )(x)


# device time: 66312 ns/iter; 1.5950x vs baseline; 1.5950x over previous
import jax
import jax.numpy as jnp
from jax import lax
from jax.experimental import pallas as pl
from jax.experimental.pallas import tpu as pltpu

N_DEV = 8

PERMS = ((1, 3, 4), (3, 4, 1), (4, 1, 3))
E_PARTNER = (0, 1, 1, 2, 2, 2, 2)
E_CHUNK = ((), (), (0,), (), (0,), (1,), (0, 1))
N_EXCH = 7

P_OFF = (0, 688, 1376)
P_LEN = (688, 688, 672)


def kernel(x):
    m_per, n = x.shape
    assert m_per == P_OFF[2] + P_LEN[2]

    def body(x_ref, out_ref, send_sems, recv_sems):
        my = lax.axis_index("i")

        barrier_sem = pltpu.get_barrier_semaphore()
        for mask in (1, 3, 4):
            pl.semaphore_signal(
                barrier_sem, inc=1,
                device_id=(my ^ mask,), device_id_type=pl.DeviceIdType.MESH,
            )
        pl.semaphore_wait(barrier_sem, 3)

        out_ref[pl.ds(my * m_per, m_per), :] = x_ref[:, :].astype(out_ref.dtype)

        send_d = {}
        recv_d = {}
        for s, perm in enumerate(PERMS):
            off, ln = P_OFF[s], P_LEN[s]

            def region(chunk, off=off, ln=ln):
                return out_ref.at[pl.ds(chunk * m_per + off, ln), :]

            for e in range(N_EXCH):
                pmask = perm[E_PARTNER[e]]
                cmask = 0
                for i in E_CHUNK[e]:
                    cmask ^= perm[i]
                k = s * N_EXCH + e
                send_chunk = my ^ cmask
                recv_chunk = my ^ cmask ^ pmask
                common = dict(
                    send_sem=send_sems.at[k],
                    recv_sem=recv_sems.at[k],
                    device_id=(my ^ pmask,),
                    device_id_type=pl.DeviceIdType.MESH,
                )
                send_d[s, e] = pltpu.make_async_remote_copy(
                    src_ref=region(send_chunk), dst_ref=region(send_chunk),
                    **common,
                )
                recv_d[s, e] = pltpu.make_async_remote_copy(
                    src_ref=region(recv_chunk), dst_ref=region(recv_chunk),
                    **common,
                )

        for s in range(3):
            send_d[s, 0].start()
        for s in range(3):
            send_d[s, 1].start()
        for s in range(3):
            send_d[s, 3].start()
        for s in range(3):
            recv_d[s, 0].wait_recv()
            send_d[s, 2].start()
            send_d[s, 4].start()
        for s in range(3):
            recv_d[s, 1].wait_recv()
            send_d[s, 5].start()
        for s in range(3):
            recv_d[s, 2].wait_recv()
            send_d[s, 6].start()
        for s in range(3):
            for e in (3, 4, 5, 6):
                recv_d[s, e].wait_recv()
        for s in range(3):
            for e in range(N_EXCH):
                send_d[s, e].wait_send()

    out_shape = jax.ShapeDtypeStruct((N_DEV * m_per, n), jnp.bfloat16)
    return pl.pallas_call(
        body,
        out_shape=out_shape,
        in_specs=[pl.BlockSpec(memory_space=pltpu.VMEM)],
        out_specs=pl.BlockSpec(memory_space=pltpu.VMEM),
        scratch_shapes=[
            pltpu.SemaphoreType.DMA((3 * N_EXCH,)),
            pltpu.SemaphoreType.DMA((3 * N_EXCH,)),
        ],
        compiler_params=pltpu.CompilerParams(collective_id=0),
    )(x)
